# baseline (speedup 1.0000x reference)
import jax
import jax.numpy as jnp
from jax import lax
from jax.experimental import pallas as pl
from jax.experimental.pallas import tpu as pltpu
from jax.experimental.pallas import tpu_sc as plsc

N = 10000
M = 10000
H = 128
S = 32

NC = 2
NS = 16
NW = NC * NS
NPW = 320

G = 1
NGRP = NPW // G
NBUF = 2


def _pre_body(pro_ref, o0_ref, o1_ref, c0_ref, c1_ref,
              a0_ref, a1_ref, s0_ref, s1_ref):
    c0 = c0_ref[...]
    c1 = c1_ref[...]
    a0_ref[...] = jnp.dot(pro_ref[...], c0[:H], preferred_element_type=jnp.float32)
    a1_ref[...] = jnp.dot(pro_ref[...], c1[:H], preferred_element_type=jnp.float32)
    s0_ref[...] = jnp.dot(o0_ref[...], c0[H:], preferred_element_type=jnp.float32)
    s1_ref[...] = jnp.dot(o1_ref[...], c1[H:], preferred_element_type=jnp.float32)


def _pre(pro, o0, o1, attn0, attn1):
    out = (
        jax.ShapeDtypeStruct((N, 1), jnp.float32),
        jax.ShapeDtypeStruct((N, 1), jnp.float32),
        jax.ShapeDtypeStruct((M, 1), jnp.float32),
        jax.ShapeDtypeStruct((M, 1), jnp.float32),
    )
    return pl.pallas_call(_pre_body, out_shape=out)(pro, o0, o1, attn0, attn1)


def _sc_body(o0_hbm, o1_hbm, idx0_hbm, idx1_hbm, a0_hbm, a1_hbm,
             s0_hbm, s1_hbm, f0_hbm, f1_hbm,
             s_v, idx_v, a_v, w_v, sc_v, rows_v, obuf0, obuf1, tab_sh,
             sem0, sem1, osem0, osem1):
    sems = (sem0, sem1)
    osems = (osem0, osem1)
    obufs = (obuf0, obuf1)
    cid = lax.axis_index("c")
    sid = lax.axis_index("s")
    wid = sid * NC + cid
    base = jnp.where(wid == NW - 1, N - NPW, wid * NPW)
    it16 = lax.iota(jnp.int32, 16)

    for path in range(2):
        table = (o0_hbm, o1_hbm)[path]
        idx_hbm = (idx0_hbm, idx1_hbm)[path]
        a_hbm = (a0_hbm, a1_hbm)[path]
        s_hbm = (s0_hbm, s1_hbm)[path]
        f_hbm = (f0_hbm, f1_hbm)[path]

        pltpu.sync_copy(s_hbm, s_v)
        pltpu.sync_copy(idx_hbm.at[pl.ds(base * S, NPW * S)], idx_v)
        pltpu.sync_copy(a_hbm.at[pl.ds(base, NPW)], a_v)

        @pl.when(sid == 0)
        def _():
            pltpu.sync_copy(table, tab_sh)

        plsc.subcore_barrier()

        def fire(g, b):
            pltpu.async_copy(
                tab_sh.at[idx_v.at[pl.ds(g * (G * S), G * S)]], rows_v.at[b],
                sems[b])

        def wait(g, b):
            pltpu.make_async_copy(
                tab_sh.at[idx_v.at[pl.ds(g * (G * S), G * S)]], rows_v.at[b],
                sems[b]).wait()

        def out_slice(g):
            return f_hbm.at[pl.ds((base + g * G) * H, G * H)]

        def fire_out(g, b):
            pltpu.async_copy(obufs[b], out_slice(g), osems[b])

        def wait_out(g, b):
            pltpu.make_async_copy(obufs[b], out_slice(g), osems[b]).wait()

        for b in range(NBUF):
            fire(b, b)

        def weights_body(q, _):
            nb = q * 16
            col0 = nb * S + S * it16
            av = a_v[pl.ds(nb, 16)]
            def score_k(k, m):
                iv = plsc.load_gather(idx_v, [col0 + k])
                sv = plsc.load_gather(s_v, [iv]) + av
                lr = jnp.where(sv >= 0.0, sv, sv * 0.01)
                sc_v[pl.ds(k * 16, 16)] = lr
                return jnp.maximum(m, lr)

            m = lax.fori_loop(0, S, score_k,
                              jnp.full((16,), -3.0e38, jnp.float32))

            def exp_k(k, d):
                e = jnp.exp(sc_v[pl.ds(k * 16, 16)] - m)
                sc_v[pl.ds(k * 16, 16)] = e
                return d + e

            dsum = lax.fori_loop(0, S, exp_k, jnp.zeros((16,), jnp.float32))
            rden = 1.0 / dsum

            def norm_k(k, _):
                wv = sc_v[pl.ds(k * 16, 16)] * rden
                plsc.store_scatter(w_v, [col0 + k], wv)
                return None

            lax.fori_loop(0, S, norm_k, None)
            return None

        lax.fori_loop(0, NPW // 16, weights_body, None)

        def group_body(i, _):
            for b in range(NBUF):
                g = i * NBUF + b
                wait(g, b)

                @pl.when(g >= NBUF)
                def _():
                    wait_out(jnp.maximum(g - NBUF, 0), b)

                for l in range(G):
                    n = g * G + l
                    wA = w_v[pl.ds(n * S, 16)]
                    wB = w_v[pl.ds(n * S + 16, 16)]
                    acc = [jnp.zeros((16,), jnp.float32) for _ in range(H // 16)]
                    for k in range(S):
                        wk = wA[k] if k < 16 else wB[k - 16]
                        for j in range(H // 16):
                            acc[j] = acc[j] + wk * rows_v[b, l * S + k, pl.ds(j * 16, 16)]
                    for j in range(H // 16):
                        obufs[b][pl.ds(l * H + j * 16, 16)] = acc[j]

                fire_out(g, b)
                gg = g + NBUF

                @pl.when(gg < NGRP)
                def _():
                    fire(gg, b)
            return None

        lax.fori_loop(0, NGRP // NBUF, group_body, None)
        for b in range(NBUF):
            wait_out(NGRP - NBUF + b, b)
        plsc.subcore_barrier()


def _sc_attend(o0, o1, idx0, idx1, a0, a1, s0, s1):
    mesh = plsc.VectorSubcoreMesh(core_axis_name="c", subcore_axis_name="s")
    fn = pl.kernel(
        _sc_body,
        out_type=(
            jax.ShapeDtypeStruct((N * H,), jnp.float32),
            jax.ShapeDtypeStruct((N * H,), jnp.float32),
        ),
        mesh=mesh,
        scratch_types=[
            pltpu.VMEM((M,), jnp.float32),
            pltpu.VMEM((NPW * S,), jnp.int32),
            pltpu.VMEM((NPW,), jnp.float32),
            pltpu.VMEM((NPW * S,), jnp.float32),
            pltpu.VMEM((S * 16,), jnp.float32),
            pltpu.VMEM((NBUF, G * S, H), jnp.float32),
            pltpu.VMEM((G * H,), jnp.float32),
            pltpu.VMEM((G * H,), jnp.float32),
            pltpu.VMEM_SHARED((M, H), jnp.float32),
            pltpu.SemaphoreType.DMA,
            pltpu.SemaphoreType.DMA,
            pltpu.SemaphoreType.DMA,
            pltpu.SemaphoreType.DMA,
        ],
        compiler_params=pltpu.CompilerParams(needs_layout_passes=False),
    )
    return fn(o0, o1, idx0, idx1, a0, a1, s0, s1)


def _post_body(f0_ref, f1_ref, wt_ref, b_ref, ba_ref, z_ref):
    f0 = f0_ref[...]
    f1 = f1_ref[...]
    wt = wt_ref[...]
    b = b_ref[...]
    ba = ba_ref[...]
    t0 = jnp.tanh(jnp.dot(f0, wt, preferred_element_type=jnp.float32) + b)
    t1 = jnp.tanh(jnp.dot(f1, wt, preferred_element_type=jnp.float32) + b)
    m0 = jnp.mean(t0, axis=0, keepdims=True)
    m1 = jnp.mean(t1, axis=0, keepdims=True)
    b0 = jnp.sum(m0 * ba, axis=1, keepdims=True)
    b1 = jnp.sum(m1 * ba, axis=1, keepdims=True)
    mx = jnp.maximum(b0, b1)
    e0 = jnp.exp(b0 - mx)
    e1 = jnp.exp(b1 - mx)
    w0 = e0 / (e0 + e1)
    w1 = e1 / (e0 + e1)
    z_ref[...] = w0 * f0 + w1 * f1


def _post(f0, f1, wt, b, ba):
    return pl.pallas_call(
        _post_body,
        out_shape=jax.ShapeDtypeStruct((N, H), jnp.float32),
    )(f0, f1, wt, b, ba)


def kernel(pro_feature, other_features_0, other_features_1,
           now_neibor_0, now_neibor_1, attn_0, attn_1, sc_W, sc_b, beta_attn):
    a0, a1, s0, s1 = _pre(pro_feature, other_features_0, other_features_1,
                          attn_0, attn_1)

    idx0 = now_neibor_0.astype(jnp.int32).reshape(-1)
    idx1 = now_neibor_1.astype(jnp.int32).reshape(-1)

    f0_flat, f1_flat = _sc_attend(other_features_0, other_features_1,
                                  idx0, idx1, a0[:, 0], a1[:, 0],
                                  s0[:, 0], s1[:, 0])
    f0 = f0_flat.reshape(N, H)
    f1 = f1_flat.reshape(N, H)

    wt = sc_W.T
    b2d = sc_b.reshape(1, H)
    ba = beta_attn.reshape(1, H)
    return _post(f0, f1, wt, b2d, ba)


# speedup vs baseline: 1.0052x; 1.0052x over previous
import jax
import jax.numpy as jnp
from jax import lax
from jax.experimental import pallas as pl
from jax.experimental.pallas import tpu as pltpu
from jax.experimental.pallas import tpu_sc as plsc

N = 10000
M = 10000
H = 128
S = 32

NC = 2
NS = 16
NW = NC * NS
NPW = 320

G = 2
NGRP = NPW // G
NBUF = 2
NH = 4
NPH = NPW // NH
NGH = NPH // G


def _pre_body(pro_ref, o0_ref, o1_ref, c0_ref, c1_ref,
              a0_ref, a1_ref, s0_ref, s1_ref):
    c0 = c0_ref[...]
    c1 = c1_ref[...]
    a0_ref[...] = jnp.dot(pro_ref[...], c0[:H], preferred_element_type=jnp.float32)
    a1_ref[...] = jnp.dot(pro_ref[...], c1[:H], preferred_element_type=jnp.float32)
    s0_ref[...] = jnp.dot(o0_ref[...], c0[H:], preferred_element_type=jnp.float32)
    s1_ref[...] = jnp.dot(o1_ref[...], c1[H:], preferred_element_type=jnp.float32)


def _pre(pro, o0, o1, attn0, attn1):
    out = (
        jax.ShapeDtypeStruct((N, 1), jnp.float32),
        jax.ShapeDtypeStruct((N, 1), jnp.float32),
        jax.ShapeDtypeStruct((M, 1), jnp.float32),
        jax.ShapeDtypeStruct((M, 1), jnp.float32),
    )
    return pl.pallas_call(_pre_body, out_shape=out)(pro, o0, o1, attn0, attn1)


def _sc_body(o0_hbm, o1_hbm, idx0_hbm, idx1_hbm, a0_hbm, a1_hbm,
             s0_hbm, s1_hbm, f0_hbm, f1_hbm,
             s_v, idx_v, a_v, w_v, sc_v, rows_v, obuf0, obuf1, tab_sh,
             sem0, sem1, osem0, osem1):
    sems = (sem0, sem1)
    osems = (osem0, osem1)
    obufs = (obuf0, obuf1)
    cid = lax.axis_index("c")
    sid = lax.axis_index("s")
    wid = sid * NC + cid
    base = jnp.where(wid == NW - 1, N - NPW, wid * NPW)
    it16 = lax.iota(jnp.int32, 16)

    for path in range(2):
        table = (o0_hbm, o1_hbm)[path]
        idx_hbm = (idx0_hbm, idx1_hbm)[path]
        a_hbm = (a0_hbm, a1_hbm)[path]
        s_hbm = (s0_hbm, s1_hbm)[path]
        f_hbm = (f0_hbm, f1_hbm)[path]

        pltpu.sync_copy(s_hbm, s_v)
        pltpu.sync_copy(idx_hbm.at[pl.ds(base * S, NPW * S)], idx_v)
        pltpu.sync_copy(a_hbm.at[pl.ds(base, NPW)], a_v)

        @pl.when(sid == 0)
        def _():
            pltpu.sync_copy(table, tab_sh)

        plsc.subcore_barrier()

        def fire(g, b):
            pltpu.async_copy(
                tab_sh.at[idx_v.at[pl.ds(g * (G * S), G * S)]], rows_v.at[b],
                sems[b])

        def wait(g, b):
            pltpu.make_async_copy(
                tab_sh.at[idx_v.at[pl.ds(g * (G * S), G * S)]], rows_v.at[b],
                sems[b]).wait()

        def out_slice(g):
            return f_hbm.at[pl.ds((base + g * G) * H, G * H)]

        def fire_out(g, b):
            pltpu.async_copy(obufs[b], out_slice(g), osems[b])

        def wait_out(g, b):
            pltpu.make_async_copy(obufs[b], out_slice(g), osems[b]).wait()

        def half_body(half, _):
            gh0 = half * NGH

            for b in range(NBUF):
                fire(gh0 + b, b)

            def weights_body(q, _):
                nb = half * NPH + q * 16
                colg = nb * S + S * it16
                coll = (q * 16) * S + S * it16
                av = a_v[pl.ds(nb, 16)]

                def score_k(k, m):
                    iv = plsc.load_gather(idx_v, [colg + k])
                    sv = plsc.load_gather(s_v, [iv]) + av
                    lr = jnp.where(sv >= 0.0, sv, sv * 0.01)
                    sc_v[pl.ds(k * 16, 16)] = lr
                    return jnp.maximum(m, lr)

                m = lax.fori_loop(0, S, score_k,
                                  jnp.full((16,), -3.0e38, jnp.float32))

                def exp_k(k, d):
                    e = jnp.exp(sc_v[pl.ds(k * 16, 16)] - m)
                    sc_v[pl.ds(k * 16, 16)] = e
                    return d + e

                dsum = lax.fori_loop(0, S, exp_k,
                                     jnp.zeros((16,), jnp.float32))
                rden = 1.0 / dsum

                def norm_k(k, _):
                    wv = sc_v[pl.ds(k * 16, 16)] * rden
                    plsc.store_scatter(w_v, [coll + k], wv)
                    return None

                lax.fori_loop(0, S, norm_k, None)
                return None

            lax.fori_loop(0, NPH // 16, weights_body, None)

            def group_body(i, _):
                for b in range(NBUF):
                    gl = i * NBUF + b
                    g = gh0 + gl
                    wait(g, b)

                    @pl.when(gl >= NBUF)
                    def _():
                        wait_out(jnp.maximum(g - NBUF, 0), b)

                    for l in range(G):
                        nh = gl * G + l
                        wA = w_v[pl.ds(nh * S, 16)]
                        wB = w_v[pl.ds(nh * S + 16, 16)]
                        acc = [jnp.zeros((16,), jnp.float32)
                               for _ in range(H // 16)]
                        for k in range(S):
                            wk = wA[k] if k < 16 else wB[k - 16]
                            for j in range(H // 16):
                                acc[j] = acc[j] + wk * rows_v[b, l * S + k, pl.ds(j * 16, 16)]
                        for j in range(H // 16):
                            obufs[b][pl.ds(l * H + j * 16, 16)] = acc[j]

                    fire_out(g, b)

                    @pl.when(gl + NBUF < NGH)
                    def _():
                        fire(g + NBUF, b)
                return None

            lax.fori_loop(0, NGH // NBUF, group_body, None)
            for b in range(NBUF):
                wait_out(gh0 + NGH - NBUF + b, b)
            return None

        lax.fori_loop(0, NH, half_body, None)
        plsc.subcore_barrier()


def _sc_attend(o0, o1, idx0, idx1, a0, a1, s0, s1):
    mesh = plsc.VectorSubcoreMesh(core_axis_name="c", subcore_axis_name="s")
    fn = pl.kernel(
        _sc_body,
        out_type=(
            jax.ShapeDtypeStruct((N * H,), jnp.float32),
            jax.ShapeDtypeStruct((N * H,), jnp.float32),
        ),
        mesh=mesh,
        scratch_types=[
            pltpu.VMEM((M,), jnp.float32),
            pltpu.VMEM((NPW * S,), jnp.int32),
            pltpu.VMEM((NPW,), jnp.float32),
            pltpu.VMEM((NPH * S,), jnp.float32),
            pltpu.VMEM((S * 16,), jnp.float32),
            pltpu.VMEM((NBUF, G * S, H), jnp.float32),
            pltpu.VMEM((G * H,), jnp.float32),
            pltpu.VMEM((G * H,), jnp.float32),
            pltpu.VMEM_SHARED((M, H), jnp.float32),
            pltpu.SemaphoreType.DMA,
            pltpu.SemaphoreType.DMA,
            pltpu.SemaphoreType.DMA,
            pltpu.SemaphoreType.DMA,
        ],
        compiler_params=pltpu.CompilerParams(needs_layout_passes=False),
    )
    return fn(o0, o1, idx0, idx1, a0, a1, s0, s1)


def _post_body(f0_ref, f1_ref, wt_ref, b_ref, ba_ref, z_ref):
    f0 = f0_ref[...]
    f1 = f1_ref[...]
    wt = wt_ref[...]
    b = b_ref[...]
    ba = ba_ref[...]
    t0 = jnp.tanh(jnp.dot(f0, wt, preferred_element_type=jnp.float32) + b)
    t1 = jnp.tanh(jnp.dot(f1, wt, preferred_element_type=jnp.float32) + b)
    m0 = jnp.mean(t0, axis=0, keepdims=True)
    m1 = jnp.mean(t1, axis=0, keepdims=True)
    b0 = jnp.sum(m0 * ba, axis=1, keepdims=True)
    b1 = jnp.sum(m1 * ba, axis=1, keepdims=True)
    mx = jnp.maximum(b0, b1)
    e0 = jnp.exp(b0 - mx)
    e1 = jnp.exp(b1 - mx)
    w0 = e0 / (e0 + e1)
    w1 = e1 / (e0 + e1)
    z_ref[...] = w0 * f0 + w1 * f1


def _post(f0, f1, wt, b, ba):
    return pl.pallas_call(
        _post_body,
        out_shape=jax.ShapeDtypeStruct((N, H), jnp.float32),
    )(f0, f1, wt, b, ba)


def kernel(pro_feature, other_features_0, other_features_1,
           now_neibor_0, now_neibor_1, attn_0, attn_1, sc_W, sc_b, beta_attn):
    a0, a1, s0, s1 = _pre(pro_feature, other_features_0, other_features_1,
                          attn_0, attn_1)

    idx0 = now_neibor_0.astype(jnp.int32).reshape(-1)
    idx1 = now_neibor_1.astype(jnp.int32).reshape(-1)

    f0_flat, f1_flat = _sc_attend(other_features_0, other_features_1,
                                  idx0, idx1, a0[:, 0], a1[:, 0],
                                  s0[:, 0], s1[:, 0])
    f0 = f0_flat.reshape(N, H)
    f1 = f1_flat.reshape(N, H)

    wt = sc_W.T
    b2d = sc_b.reshape(1, H)
    ba = beta_attn.reshape(1, H)
    return _post(f0, f1, wt, b2d, ba)


# speedup vs baseline: 1.3356x; 1.3287x over previous
import jax
import jax.numpy as jnp
from jax import lax
from jax.experimental import pallas as pl
from jax.experimental.pallas import tpu as pltpu
from jax.experimental.pallas import tpu_sc as plsc

N = 10000
M = 10000
H = 128
S = 32

NC = 2
NS = 16
NW = NC * NS
NPW = 320

G = 2
NGRP = NPW // G
NBUF = 2


def _pre_body(pro_ref, o0_ref, o1_ref, c0_ref, c1_ref,
              a0_ref, a1_ref, s0_ref, s1_ref):
    c0 = c0_ref[...]
    c1 = c1_ref[...]
    a0_ref[...] = jnp.dot(pro_ref[...], c0[:H], preferred_element_type=jnp.float32)
    a1_ref[...] = jnp.dot(pro_ref[...], c1[:H], preferred_element_type=jnp.float32)
    s0_ref[...] = jnp.dot(o0_ref[...], c0[H:], preferred_element_type=jnp.float32)
    s1_ref[...] = jnp.dot(o1_ref[...], c1[H:], preferred_element_type=jnp.float32)


def _pre(pro, o0, o1, attn0, attn1):
    out = (
        jax.ShapeDtypeStruct((N, 1), jnp.float32),
        jax.ShapeDtypeStruct((N, 1), jnp.float32),
        jax.ShapeDtypeStruct((M, 1), jnp.float32),
        jax.ShapeDtypeStruct((M, 1), jnp.float32),
    )
    return pl.pallas_call(_pre_body, out_shape=out)(pro, o0, o1, attn0, attn1)


def _sc_body(o0_hbm, o1_hbm, idx0_hbm, idx1_hbm, a0_hbm, a1_hbm,
             s0_hbm, s1_hbm, f0_hbm, f1_hbm,
             s_v, idx_v, a_v, rows_v, obuf0, obuf1, tab_sh,
             sem0, sem1, osem0, osem1):
    sems = (sem0, sem1)
    osems = (osem0, osem1)
    obufs = (obuf0, obuf1)
    cid = lax.axis_index("c")
    sid = lax.axis_index("s")
    wid = sid * NC + cid
    base = jnp.where(wid == NW - 1, N - NPW, wid * NPW)

    for path in range(2):
        table = (o0_hbm, o1_hbm)[path]
        idx_hbm = (idx0_hbm, idx1_hbm)[path]
        a_hbm = (a0_hbm, a1_hbm)[path]
        s_hbm = (s0_hbm, s1_hbm)[path]
        f_hbm = (f0_hbm, f1_hbm)[path]

        pltpu.sync_copy(s_hbm, s_v)
        pltpu.sync_copy(idx_hbm.at[pl.ds(base * S, NPW * S)], idx_v)
        pltpu.sync_copy(a_hbm.at[pl.ds(base, NPW)], a_v.at[pl.ds(0, NPW)])

        @pl.when(sid == 0)
        def _():
            pltpu.sync_copy(table, tab_sh)

        plsc.subcore_barrier()

        def fire(g, b):
            pltpu.async_copy(
                tab_sh.at[idx_v.at[pl.ds(g * (G * S), G * S)]], rows_v.at[b],
                sems[b])

        def wait(g, b):
            pltpu.make_async_copy(
                tab_sh.at[idx_v.at[pl.ds(g * (G * S), G * S)]], rows_v.at[b],
                sems[b]).wait()

        def out_slice(g):
            return f_hbm.at[pl.ds((base + g * G) * H, G * H)]

        def fire_out(g, b):
            pltpu.async_copy(obufs[b], out_slice(g), osems[b])

        def wait_out(g, b):
            pltpu.make_async_copy(obufs[b], out_slice(g), osems[b]).wait()

        for b in range(NBUF):
            fire(b, b)

        def group_body(i, _):
            for b in range(NBUF):
                g = i * NBUF + b
                wait(g, b)

                @pl.when(g >= NBUF)
                def _():
                    wait_out(jnp.maximum(g - NBUF, 0), b)

                for l in range(G):
                    n = g * G + l
                    idxA = idx_v[pl.ds(n * S, 16)]
                    idxB = idx_v[pl.ds(n * S + 16, 16)]
                    a = a_v[pl.ds(n, 16)][0]
                    sA = plsc.load_gather(s_v, [idxA]) + a
                    sB = plsc.load_gather(s_v, [idxB]) + a
                    lrA = jnp.where(sA >= 0.0, sA, sA * 0.01)
                    lrB = jnp.where(sB >= 0.0, sB, sB * 0.01)
                    m = jnp.max(jnp.maximum(lrA, lrB))
                    eA = jnp.exp(lrA - m)
                    eB = jnp.exp(lrB - m)
                    denom = jnp.sum(eA + eB)
                    wA = eA / denom
                    wB = eB / denom

                    acc = [jnp.zeros((16,), jnp.float32) for _ in range(H // 16)]
                    for k in range(S):
                        wk = wA[k] if k < 16 else wB[k - 16]
                        for j in range(H // 16):
                            acc[j] = acc[j] + wk * rows_v[b, l * S + k, pl.ds(j * 16, 16)]
                    for j in range(H // 16):
                        obufs[b][pl.ds(l * H + j * 16, 16)] = acc[j]

                fire_out(g, b)
                gg = g + NBUF

                @pl.when(gg < NGRP)
                def _():
                    fire(gg, b)
            return None

        lax.fori_loop(0, NGRP // NBUF, group_body, None)
        for b in range(NBUF):
            wait_out(NGRP - NBUF + b, b)
        plsc.subcore_barrier()


def _sc_attend(o0, o1, idx0, idx1, a0, a1, s0, s1):
    mesh = plsc.VectorSubcoreMesh(core_axis_name="c", subcore_axis_name="s")
    fn = pl.kernel(
        _sc_body,
        out_type=(
            jax.ShapeDtypeStruct((N * H,), jnp.float32),
            jax.ShapeDtypeStruct((N * H,), jnp.float32),
        ),
        mesh=mesh,
        scratch_types=[
            pltpu.VMEM((M,), jnp.float32),
            pltpu.VMEM((NPW * S,), jnp.int32),
            pltpu.VMEM((NPW + 16,), jnp.float32),
            pltpu.VMEM((NBUF, G * S, H), jnp.float32),
            pltpu.VMEM((G * H,), jnp.float32),
            pltpu.VMEM((G * H,), jnp.float32),
            pltpu.VMEM_SHARED((M, H), jnp.float32),
            pltpu.SemaphoreType.DMA,
            pltpu.SemaphoreType.DMA,
            pltpu.SemaphoreType.DMA,
            pltpu.SemaphoreType.DMA,
        ],
        compiler_params=pltpu.CompilerParams(needs_layout_passes=False),
    )
    return fn(o0, o1, idx0, idx1, a0, a1, s0, s1)


def _post_body(f0_ref, f1_ref, wt_ref, b_ref, ba_ref, z_ref):
    f0 = f0_ref[...]
    f1 = f1_ref[...]
    wt = wt_ref[...]
    b = b_ref[...]
    ba = ba_ref[...]
    t0 = jnp.tanh(jnp.dot(f0, wt, preferred_element_type=jnp.float32) + b)
    t1 = jnp.tanh(jnp.dot(f1, wt, preferred_element_type=jnp.float32) + b)
    m0 = jnp.mean(t0, axis=0, keepdims=True)
    m1 = jnp.mean(t1, axis=0, keepdims=True)
    b0 = jnp.sum(m0 * ba, axis=1, keepdims=True)
    b1 = jnp.sum(m1 * ba, axis=1, keepdims=True)
    mx = jnp.maximum(b0, b1)
    e0 = jnp.exp(b0 - mx)
    e1 = jnp.exp(b1 - mx)
    w0 = e0 / (e0 + e1)
    w1 = e1 / (e0 + e1)
    z_ref[...] = w0 * f0 + w1 * f1


def _post(f0, f1, wt, b, ba):
    return pl.pallas_call(
        _post_body,
        out_shape=jax.ShapeDtypeStruct((N, H), jnp.float32),
    )(f0, f1, wt, b, ba)


def kernel(pro_feature, other_features_0, other_features_1,
           now_neibor_0, now_neibor_1, attn_0, attn_1, sc_W, sc_b, beta_attn):
    a0, a1, s0, s1 = _pre(pro_feature, other_features_0, other_features_1,
                          attn_0, attn_1)

    idx0 = now_neibor_0.astype(jnp.int32).reshape(-1)
    idx1 = now_neibor_1.astype(jnp.int32).reshape(-1)

    f0_flat, f1_flat = _sc_attend(other_features_0, other_features_1,
                                  idx0, idx1, a0[:, 0], a1[:, 0],
                                  s0[:, 0], s1[:, 0])
    f0 = f0_flat.reshape(N, H)
    f1 = f1_flat.reshape(N, H)

    wt = sc_W.T
    b2d = sc_b.reshape(1, H)
    ba = beta_attn.reshape(1, H)
    return _post(f0, f1, wt, b2d, ba)
